# baseline (speedup 1.0000x reference)
import jax
import jax.numpy as jnp
from jax import lax
from jax.experimental import pallas as pl

_V = 1000
_E = 16
_B = 1024
_NR = _V * _E
_RB = 400


def _fill_kernel(i0, i1, i2, i3, w0, w1, w2, w3, o0, o1, o2, o3):
    rowv = lax.shift_right_logical(
        pl.program_id(0) * _RB
        + lax.broadcasted_iota(jnp.int32, (_RB, _B), 0),
        4,
    )
    for i_ref, w_ref, o_ref in ((i0, w0, o0), (i1, w1, o1), (i2, w2, o2), (i3, w3, o3)):
        mask = rowv == i_ref[0]
        o_ref[...] = jnp.where(mask, w_ref[:, 1:2], w_ref[:, 0:1])


def kernel(input0, input1, input2, input3, table0, table1, table2, table3):
    inputs = (input0, input1, input2, input3)
    tables = (table0, table1, table2, table3)

    idx = [inp.reshape(1, 1, _B) for inp in inputs]
    cols = [jnp.stack([jnp.tile(t[0], _V), jnp.tile(t[1], _V)], axis=1)
            for t in tables]

    grid = (_NR // _RB,)
    in_specs = (
        [pl.BlockSpec((1, 1, _B), lambda i: (0, 0, 0)) for _ in range(4)]
        + [pl.BlockSpec((_RB, 2), lambda i: (i, 0)) for _ in range(4)]
    )
    out_specs = [pl.BlockSpec((_RB, _B), lambda i: (i, 0)) for _ in range(4)]
    outs = pl.pallas_call(
        _fill_kernel,
        grid=grid,
        in_specs=in_specs,
        out_specs=out_specs,
        out_shape=[jax.ShapeDtypeStruct((_NR, _B), jnp.float32)] * 4,
    )(*idx, *cols)
    return tuple(
        o.reshape(_V, _E, _B).transpose(2, 0, 1) for o in outs
    )


# speedup vs baseline: 1.0226x; 1.0226x over previous
import jax
import jax.numpy as jnp
from jax import lax
from jax.experimental import pallas as pl

_V = 1000
_E = 16
_B = 1024
_NR = _V * _E
_RB = 1000


def _fill_kernel(i0, i1, i2, i3, w0, w1, w2, w3, o0, o1, o2, o3):
    rowv = lax.shift_right_logical(
        pl.program_id(0) * _RB
        + lax.broadcasted_iota(jnp.int32, (_RB, _B), 0),
        4,
    )
    for i_ref, w_ref, o_ref in ((i0, w0, o0), (i1, w1, o1), (i2, w2, o2), (i3, w3, o3)):
        mask = rowv == i_ref[0]
        o_ref[...] = jnp.where(mask, w_ref[:, 1:2], w_ref[:, 0:1])


def kernel(input0, input1, input2, input3, table0, table1, table2, table3):
    inputs = (input0, input1, input2, input3)
    tables = (table0, table1, table2, table3)

    idx = [inp.reshape(1, 1, _B) for inp in inputs]
    cols = [jnp.stack([jnp.tile(t[0], _V), jnp.tile(t[1], _V)], axis=1)
            for t in tables]

    grid = (_NR // _RB,)
    in_specs = (
        [pl.BlockSpec((1, 1, _B), lambda i: (0, 0, 0)) for _ in range(4)]
        + [pl.BlockSpec((_RB, 2), lambda i: (i, 0)) for _ in range(4)]
    )
    out_specs = [pl.BlockSpec((_RB, _B), lambda i: (i, 0)) for _ in range(4)]
    outs = pl.pallas_call(
        _fill_kernel,
        grid=grid,
        in_specs=in_specs,
        out_specs=out_specs,
        out_shape=[jax.ShapeDtypeStruct((_NR, _B), jnp.float32)] * 4,
    )(*idx, *cols)
    return tuple(
        o.reshape(_V, _E, _B).transpose(2, 0, 1) for o in outs
    )
